# baseline (speedup 1.0000x reference)
import dataclasses
import functools

import jax
import jax.numpy as jnp
from jax import lax
from jax.experimental import pallas as pl
from jax.experimental.pallas import tpu as pltpu
from jax.experimental.pallas import tpu_sc as plsc

_NPAD = 10240
_NTILES = 16
_NCORES = 2
_RPT = _NPAD // _NTILES
_K = 128
_D = 128


def _mesh():
    return plsc.VectorSubcoreMesh(core_axis_name="c", subcore_axis_name="s")


def _sc_params():
    cp = pltpu.CompilerParams()
    if "needs_layout_passes" in pltpu.CompilerParams.__dataclass_fields__:
        cp = dataclasses.replace(cp, needs_layout_passes=False)
    return cp


def _make_hop(nblk, width, with_r):
    assert nblk % 2 == 0
    outs = [jax.ShapeDtypeStruct((_NCORES, _NPAD, width), jnp.float32)]
    scratch = [
        pltpu.VMEM((2, 2, _K), jnp.int32),
        pltpu.VMEM((_K, width), jnp.float32),
        pltpu.VMEM((_K, width), jnp.float32),
        pltpu.VMEM_SHARED((_NPAD, width), jnp.float32),
        pltpu.SemaphoreType.DMA,
        pltpu.SemaphoreType.DMA,
        pltpu.SemaphoreType.DMA,
        pltpu.SemaphoreType.DMA,
    ]
    if with_r:
        outs.append(jax.ShapeDtypeStruct((_NCORES, _NTILES, _NPAD),
                                         jnp.float32))
        scratch += [
            pltpu.VMEM((2, _K), jnp.float32),
            pltpu.VMEM((_NPAD,), jnp.float32),
            pltpu.SemaphoreType.DMA,
            pltpu.SemaphoreType.DMA,
        ]

    @functools.partial(
        pl.kernel,
        out_type=tuple(outs) if with_r else outs[0],
        mesh=_mesh(),
        scratch_types=scratch,
        compiler_params=_sc_params(),
    )
    def hop(table_hbm, idx_hbm, sv_hbm, zero_hbm, zero1_hbm, *refs):
        if with_r:
            (main_out, rout, idx_v, rows0, rows1, acc_sh, sem0, sem1,
             semi0, semi1, sv_v, racc_v, sems0, sems1) = refs
        else:
            (main_out, idx_v, rows0, rows1, acc_sh, sem0, sem1,
             semi0, semi1) = refs
        cid = lax.axis_index("c")
        sid = lax.axis_index("s")
        pltpu.sync_copy(zero_hbm.at[pl.ds(sid * _RPT, _RPT)],
                        acc_sh.at[pl.ds(sid * _RPT, _RPT)])
        if with_r:
            pltpu.sync_copy(zero1_hbm, racc_v)
        base_blk = (cid * _NTILES + sid) * nblk
        plsc.subcore_barrier()

        def scalar_hop(p):
            @pl.loop(0, _K // 16)
            def _(j):
                d16 = idx_v[p, 1, pl.ds(j * 16, 16)]
                vals = sv_v[p, pl.ds(j * 16, 16)]
                plsc.addupdate_scatter(racc_v, [d16], vals)

        pltpu.sync_copy(idx_hbm.at[base_blk], idx_v.at[0])
        pltpu.async_copy(idx_hbm.at[base_blk + 1], idx_v.at[1], semi1)
        if with_r:
            pltpu.async_copy(sv_hbm.at[base_blk], sv_v.at[0], sems0)
        pltpu.async_copy(table_hbm.at[idx_v.at[0, 0]], rows0, sem0)

        @pl.loop(0, nblk // 2)
        def _(i):
            b0 = i * 2
            pltpu.make_async_copy(idx_hbm.at[base_blk], idx_v.at[1],
                                  semi1).wait()
            pltpu.async_copy(table_hbm.at[idx_v.at[1, 0]], rows1, sem1)
            if with_r:
                pltpu.async_copy(sv_hbm.at[base_blk + b0 + 1], sv_v.at[1],
                                 sems1)
            pltpu.make_async_copy(table_hbm.at[idx_v.at[0, 0]],
                                  rows0, sem0).wait()
            pltpu.sync_copy(rows0, acc_sh.at[idx_v.at[0, 1]], add=True)
            if with_r:
                pltpu.make_async_copy(sv_hbm.at[base_blk], sv_v.at[0],
                                      sems0).wait()
                scalar_hop(0)

            @pl.when(b0 + 2 < nblk)
            def _():
                pltpu.async_copy(idx_hbm.at[base_blk + b0 + 2], idx_v.at[0],
                                 semi0)
                pltpu.make_async_copy(idx_hbm.at[base_blk], idx_v.at[0],
                                      semi0).wait()
                pltpu.async_copy(table_hbm.at[idx_v.at[0, 0]], rows0, sem0)
                if with_r:
                    pltpu.async_copy(sv_hbm.at[base_blk + b0 + 2], sv_v.at[0],
                                     sems0)

            pltpu.make_async_copy(table_hbm.at[idx_v.at[1, 0]],
                                  rows1, sem1).wait()
            pltpu.sync_copy(rows1, acc_sh.at[idx_v.at[1, 1]], add=True)
            if with_r:
                pltpu.make_async_copy(sv_hbm.at[base_blk], sv_v.at[1],
                                      sems1).wait()
                scalar_hop(1)

            @pl.when(b0 + 3 < nblk)
            def _():
                pltpu.async_copy(idx_hbm.at[base_blk + b0 + 3], idx_v.at[1],
                                 semi1)

        plsc.subcore_barrier()
        pltpu.sync_copy(acc_sh.at[pl.ds(sid * _RPT, _RPT)],
                        main_out.at[cid, pl.ds(sid * _RPT, _RPT)])
        if with_r:
            pltpu.sync_copy(racc_v, rout.at[cid, sid])

    return hop


def _make_hist(nblk):
    @functools.partial(
        pl.kernel,
        out_type=jax.ShapeDtypeStruct((_NCORES, _NTILES, _NPAD), jnp.float32),
        mesh=_mesh(),
        scratch_types=[
            pltpu.VMEM((2, _K), jnp.int32),
            pltpu.VMEM((_NPAD,), jnp.float32),
        ],
        compiler_params=_sc_params(),
    )
    def hist(idx_hbm, out_hbm, idx_v, deg_v):
        cid = lax.axis_index("c")
        sid = lax.axis_index("s")

        @pl.loop(0, _NPAD // 16)
        def _(i):
            deg_v[pl.ds(i * 16, 16)] = jnp.zeros((16,), jnp.float32)

        base_blk = (cid * _NTILES + sid) * nblk
        ones16 = jnp.full((16,), 1.0, jnp.float32)

        @pl.loop(0, nblk)
        def _(b):
            pltpu.sync_copy(idx_hbm.at[base_blk + b], idx_v)

            @pl.loop(0, _K // 16)
            def _(j):
                d16 = idx_v[1, pl.ds(j * 16, 16)]
                plsc.addupdate_scatter(deg_v, [d16], ones16)

        pltpu.sync_copy(deg_v, out_hbm.at[cid, sid])

    return hist


def _final_tc_kernel(z_ref, r_ref, w0_ref, w1_ref, wout_ref, b0_ref, b1_ref,
                     bout_ref, vn_ref, out_ref):
    hi = jax.lax.Precision.HIGHEST
    w1wout = jax.lax.dot_general(
        w1_ref[...], wout_ref[...], (((1,), (0,)), ((), ())),
        preferred_element_type=jnp.float32, precision=hi)
    wc = jax.lax.dot_general(
        w0_ref[...], w1wout, (((1,), (0,)), ((), ())),
        preferred_element_type=jnp.float32, precision=hi)
    c0 = b0_ref[...] + vn_ref[...]
    cvec = jax.lax.dot_general(c0, w1wout, (((1,), (0,)), ((), ())),
                               preferred_element_type=jnp.float32, precision=hi)
    bvec = jax.lax.dot_general(b1_ref[...], wout_ref[...],
                               (((1,), (0,)), ((), ())),
                               preferred_element_type=jnp.float32,
                               precision=hi) + bout_ref[...]
    y = jax.lax.dot_general(z_ref[...], wc, (((1,), (0,)), ((), ())),
                            preferred_element_type=jnp.float32, precision=hi)
    out_ref[...] = y + r_ref[...] * cvec + bvec


def _final_matmul(z, r, W0, W1, Wout, b0, b1, bout, vn):
    n, d_out = z.shape[0], Wout.shape[1]
    return pl.pallas_call(
        _final_tc_kernel,
        out_shape=jax.ShapeDtypeStruct((n, d_out), jnp.float32),
    )(z, r.reshape(n, 1), W0, W1, Wout, b0.reshape(1, -1), b1.reshape(1, -1),
      bout.reshape(1, -1), vn)


def kernel(x, edge_index, W0, b0, W1, b1, vn, mW1, mb1, mW2, mb2, Wout, bout):
    n, d = x.shape
    src, dst = edge_index[0], edge_index[1]
    e = src.shape[0]
    nw = _NCORES * _NTILES
    nblk = -(-e // (nw * _K))
    nblk += nblk % 2
    epw = nblk * _K
    npad_edges = nw * epw - e
    pad_idx = (n + (jnp.arange(npad_edges) % (_NPAD - n))).astype(jnp.int32)
    srcs = jnp.concatenate([src, pad_idx])
    dsts = jnp.concatenate([dst, pad_idx])
    idx3 = jnp.stack([srcs.reshape(-1, _K), dsts.reshape(-1, _K)], axis=1)
    zero_tab = jnp.zeros((_NPAD, _D), jnp.float32)

    parts = _make_hist(nblk)(idx3)
    deg = jnp.sum(parts, axis=(0, 1))[:n] + 1.0
    dinv = lax.rsqrt(deg)

    dinv_pad = jnp.zeros((_NPAD,), jnp.float32).at[:n].set(dinv)
    svals = dinv_pad[srcs].reshape(-1, _K)
    zero1 = jnp.zeros((_NPAD,), jnp.float32)
    u = x * dinv[:, None]
    u_pad = jnp.zeros((_NPAD, d), jnp.float32).at[:n].set(u)
    p1, racc = _make_hop(nblk, _D, True)(u_pad, idx3, svals, zero_tab, zero1)
    m = p1[0, :n] + p1[1, :n] + u
    r = dinv * (jnp.sum(racc, axis=(0, 1))[:n] + dinv)
    w = m * (dinv * dinv)[:, None]

    w_pad = jnp.zeros((_NPAD, d), jnp.float32).at[:n].set(w)
    p2 = _make_hop(nblk, _D, False)(w_pad, idx3, svals, zero_tab, zero1)
    z = (p2[0, :n] + p2[1, :n] + w) * dinv[:, None]

    return _final_matmul(z, r, W0, W1, Wout, b0, b1, bout, vn)


# speedup vs baseline: 10.1298x; 10.1298x over previous
import dataclasses
import functools

import jax
import jax.numpy as jnp
from jax import lax
from jax.experimental import pallas as pl
from jax.experimental.pallas import tpu as pltpu
from jax.experimental.pallas import tpu_sc as plsc

_NPAD = 10240
_NTILES = 16
_NCORES = 2
_RPT = _NPAD // _NTILES
_K = 128
_D = 128


def _mesh():
    return plsc.VectorSubcoreMesh(core_axis_name="c", subcore_axis_name="s")


def _sc_params():
    cp = pltpu.CompilerParams()
    if "needs_layout_passes" in pltpu.CompilerParams.__dataclass_fields__:
        cp = dataclasses.replace(cp, needs_layout_passes=False)
    return cp


def _make_hop(nblk, width):
    assert nblk % 2 == 0

    @functools.partial(
        pl.kernel,
        out_type=jax.ShapeDtypeStruct((_NCORES, _NPAD, width), jnp.float32),
        mesh=_mesh(),
        scratch_types=[
            pltpu.VMEM((2, 2, _K), jnp.int32),
            pltpu.VMEM((_K, width), jnp.float32),
            pltpu.VMEM((_K, width), jnp.float32),
            pltpu.VMEM_SHARED((_NPAD, width), jnp.float32),
            pltpu.SemaphoreType.DMA,
            pltpu.SemaphoreType.DMA,
            pltpu.SemaphoreType.DMA,
            pltpu.SemaphoreType.DMA,
        ],
        compiler_params=_sc_params(),
    )
    def hop(table_hbm, idx_hbm, zero_hbm, main_out, idx_v, rows0, rows1,
            acc_sh, sem0, sem1, semi0, semi1):
        cid = lax.axis_index("c")
        sid = lax.axis_index("s")
        pltpu.sync_copy(zero_hbm.at[pl.ds(sid * _RPT, _RPT)],
                        acc_sh.at[pl.ds(sid * _RPT, _RPT)])
        base_blk = (cid * _NTILES + sid) * nblk
        plsc.subcore_barrier()

        pltpu.sync_copy(idx_hbm.at[base_blk], idx_v.at[0])
        pltpu.async_copy(idx_hbm.at[base_blk + 1], idx_v.at[1], semi1)
        pltpu.async_copy(table_hbm.at[idx_v.at[0, 0]], rows0, sem0)

        @pl.loop(0, nblk // 2)
        def _(i):
            b0 = i * 2
            pltpu.make_async_copy(idx_hbm.at[base_blk], idx_v.at[1],
                                  semi1).wait()
            pltpu.async_copy(table_hbm.at[idx_v.at[1, 0]], rows1, sem1)
            pltpu.make_async_copy(table_hbm.at[idx_v.at[0, 0]],
                                  rows0, sem0).wait()
            pltpu.sync_copy(rows0, acc_sh.at[idx_v.at[0, 1]], add=True)

            @pl.when(b0 + 2 < nblk)
            def _():
                pltpu.async_copy(idx_hbm.at[base_blk + b0 + 2], idx_v.at[0],
                                 semi0)
                pltpu.make_async_copy(idx_hbm.at[base_blk], idx_v.at[0],
                                      semi0).wait()
                pltpu.async_copy(table_hbm.at[idx_v.at[0, 0]], rows0, sem0)

            pltpu.make_async_copy(table_hbm.at[idx_v.at[1, 0]],
                                  rows1, sem1).wait()
            pltpu.sync_copy(rows1, acc_sh.at[idx_v.at[1, 1]], add=True)

            @pl.when(b0 + 3 < nblk)
            def _():
                pltpu.async_copy(idx_hbm.at[base_blk + b0 + 3], idx_v.at[1],
                                 semi1)

        plsc.subcore_barrier()
        pltpu.sync_copy(acc_sh.at[pl.ds(sid * _RPT, _RPT)],
                        main_out.at[cid, pl.ds(sid * _RPT, _RPT)])

    return hop


def _make_rprop(nblk):
    @functools.partial(
        pl.kernel,
        out_type=jax.ShapeDtypeStruct((_NCORES, _NTILES, _NPAD), jnp.float32),
        mesh=_mesh(),
        scratch_types=[
            pltpu.VMEM((2, _K), jnp.int32),
            pltpu.VMEM((_NPAD,), jnp.float32),
            pltpu.VMEM((_NPAD,), jnp.float32),
        ],
        compiler_params=_sc_params(),
    )
    def rprop(dinv_hbm, idx_hbm, out_hbm, idx_v, dinv_v, racc_v):
        cid = lax.axis_index("c")
        sid = lax.axis_index("s")
        pltpu.sync_copy(dinv_hbm, dinv_v)

        @pl.loop(0, _NPAD // 16)
        def _(i):
            racc_v[pl.ds(i * 16, 16)] = jnp.zeros((16,), jnp.float32)

        base_blk = (cid * _NTILES + sid) * nblk

        @pl.loop(0, nblk)
        def _(b):
            pltpu.sync_copy(idx_hbm.at[base_blk + b], idx_v)

            @pl.loop(0, _K // 16)
            def _(j):
                s16 = idx_v[0, pl.ds(j * 16, 16)]
                d16 = idx_v[1, pl.ds(j * 16, 16)]
                vals = plsc.load_gather(dinv_v, [s16])
                plsc.addupdate_scatter(racc_v, [d16], vals)

        pltpu.sync_copy(racc_v, out_hbm.at[cid, sid])

    return rprop


def _make_hist(nblk):
    @functools.partial(
        pl.kernel,
        out_type=jax.ShapeDtypeStruct((_NCORES, _NTILES, _NPAD), jnp.float32),
        mesh=_mesh(),
        scratch_types=[
            pltpu.VMEM((2, _K), jnp.int32),
            pltpu.VMEM((_NPAD,), jnp.float32),
        ],
        compiler_params=_sc_params(),
    )
    def hist(idx_hbm, out_hbm, idx_v, deg_v):
        cid = lax.axis_index("c")
        sid = lax.axis_index("s")

        @pl.loop(0, _NPAD // 16)
        def _(i):
            deg_v[pl.ds(i * 16, 16)] = jnp.zeros((16,), jnp.float32)

        base_blk = (cid * _NTILES + sid) * nblk
        ones16 = jnp.full((16,), 1.0, jnp.float32)

        @pl.loop(0, nblk)
        def _(b):
            pltpu.sync_copy(idx_hbm.at[base_blk + b], idx_v)

            @pl.loop(0, _K // 16)
            def _(j):
                d16 = idx_v[1, pl.ds(j * 16, 16)]
                plsc.addupdate_scatter(deg_v, [d16], ones16)

        pltpu.sync_copy(deg_v, out_hbm.at[cid, sid])

    return hist


def _final_tc_kernel(z_ref, r_ref, w0_ref, w1_ref, wout_ref, b0_ref, b1_ref,
                     bout_ref, vn_ref, out_ref):
    hi = jax.lax.Precision.HIGHEST
    w1wout = jax.lax.dot_general(
        w1_ref[...], wout_ref[...], (((1,), (0,)), ((), ())),
        preferred_element_type=jnp.float32, precision=hi)
    wc = jax.lax.dot_general(
        w0_ref[...], w1wout, (((1,), (0,)), ((), ())),
        preferred_element_type=jnp.float32, precision=hi)
    c0 = b0_ref[...] + vn_ref[...]
    cvec = jax.lax.dot_general(c0, w1wout, (((1,), (0,)), ((), ())),
                               preferred_element_type=jnp.float32, precision=hi)
    bvec = jax.lax.dot_general(b1_ref[...], wout_ref[...],
                               (((1,), (0,)), ((), ())),
                               preferred_element_type=jnp.float32,
                               precision=hi) + bout_ref[...]
    y = jax.lax.dot_general(z_ref[...], wc, (((1,), (0,)), ((), ())),
                            preferred_element_type=jnp.float32, precision=hi)
    out_ref[...] = y + r_ref[...] * cvec + bvec


def _final_matmul(z, r, W0, W1, Wout, b0, b1, bout, vn):
    n, d_out = z.shape[0], Wout.shape[1]
    return pl.pallas_call(
        _final_tc_kernel,
        out_shape=jax.ShapeDtypeStruct((n, d_out), jnp.float32),
    )(z, r.reshape(n, 1), W0, W1, Wout, b0.reshape(1, -1), b1.reshape(1, -1),
      bout.reshape(1, -1), vn)


def kernel(x, edge_index, W0, b0, W1, b1, vn, mW1, mb1, mW2, mb2, Wout, bout):
    n, d = x.shape
    src, dst = edge_index[0], edge_index[1]
    e = src.shape[0]
    nw = _NCORES * _NTILES
    nblk = -(-e // (nw * _K))
    nblk += nblk % 2
    epw = nblk * _K
    npad_edges = nw * epw - e
    pad_idx = (n + (jnp.arange(npad_edges) % (_NPAD - n))).astype(jnp.int32)
    srcs = jnp.concatenate([src, pad_idx])
    dsts = jnp.concatenate([dst, pad_idx])
    idx3 = jnp.stack([srcs.reshape(-1, _K), dsts.reshape(-1, _K)], axis=1)
    zero_tab = jnp.zeros((_NPAD, _D), jnp.float32)

    parts = _make_hist(nblk)(idx3)
    deg = jnp.sum(parts, axis=(0, 1))[:n] + 1.0
    dinv = lax.rsqrt(deg)

    dinv_pad = jnp.zeros((_NPAD,), jnp.float32).at[:n].set(dinv)
    u = x * dinv[:, None]
    u_pad = jnp.zeros((_NPAD, d), jnp.float32).at[:n].set(u)
    p1 = _make_hop(nblk, _D)(u_pad, idx3, zero_tab)
    rparts = _make_rprop(nblk)(dinv_pad, idx3)
    m = p1[0, :n] + p1[1, :n] + u
    r = dinv * (jnp.sum(rparts, axis=(0, 1))[:n] + dinv)
    w = m * (dinv * dinv)[:, None]

    w_pad = jnp.zeros((_NPAD, d), jnp.float32).at[:n].set(w)
    p2 = _make_hop(nblk, _D)(w_pad, idx3, zero_tab)
    z = (p2[0, :n] + p2[1, :n] + w) * dinv[:, None]

    return _final_matmul(z, r, W0, W1, Wout, b0, b1, bout, vn)


# speedup vs baseline: 11.2821x; 1.1137x over previous
import dataclasses
import functools

import jax
import jax.numpy as jnp
from jax import lax
from jax.experimental import pallas as pl
from jax.experimental.pallas import tpu as pltpu
from jax.experimental.pallas import tpu_sc as plsc

_NPAD = 10240
_NTILES = 16
_NCORES = 2
_RPT = _NPAD // _NTILES
_K = 128
_D = 128


def _mesh():
    return plsc.VectorSubcoreMesh(core_axis_name="c", subcore_axis_name="s")


def _sc_params():
    cp = pltpu.CompilerParams()
    if "needs_layout_passes" in pltpu.CompilerParams.__dataclass_fields__:
        cp = dataclasses.replace(cp, needs_layout_passes=False)
    return cp


def _make_hop(nblk, width):
    assert nblk % 2 == 0

    @functools.partial(
        pl.kernel,
        out_type=jax.ShapeDtypeStruct((_NCORES, _NPAD, width), jnp.float32),
        mesh=_mesh(),
        scratch_types=[
            pltpu.VMEM((2, 2, _K), jnp.int32),
            pltpu.VMEM((_K, width), jnp.float32),
            pltpu.VMEM((_K, width), jnp.float32),
            pltpu.VMEM_SHARED((_NPAD, width), jnp.float32),
            pltpu.SemaphoreType.DMA,
            pltpu.SemaphoreType.DMA,
            pltpu.SemaphoreType.DMA,
            pltpu.SemaphoreType.DMA,
        ],
        compiler_params=_sc_params(),
    )
    def hop(table_hbm, idx_hbm, zero_hbm, main_out, idx_v, rows0, rows1,
            acc_sh, sem0, sem1, semi0, semi1):
        cid = lax.axis_index("c")
        sid = lax.axis_index("s")
        pltpu.sync_copy(zero_hbm.at[pl.ds(sid * _RPT, _RPT)],
                        acc_sh.at[pl.ds(sid * _RPT, _RPT)])
        base_blk = (cid * _NTILES + sid) * nblk
        plsc.subcore_barrier()

        pltpu.sync_copy(idx_hbm.at[base_blk], idx_v.at[0])
        pltpu.async_copy(idx_hbm.at[base_blk + 1], idx_v.at[1], semi1)
        pltpu.async_copy(table_hbm.at[idx_v.at[0, 0]], rows0, sem0)

        @pl.loop(0, nblk // 2)
        def _(i):
            b0 = i * 2
            pltpu.make_async_copy(idx_hbm.at[base_blk], idx_v.at[1],
                                  semi1).wait()
            pltpu.async_copy(table_hbm.at[idx_v.at[1, 0]], rows1, sem1)
            pltpu.make_async_copy(table_hbm.at[idx_v.at[0, 0]],
                                  rows0, sem0).wait()
            pltpu.sync_copy(rows0, acc_sh.at[idx_v.at[0, 1]], add=True)

            @pl.when(b0 + 2 < nblk)
            def _():
                pltpu.async_copy(idx_hbm.at[base_blk + b0 + 2], idx_v.at[0],
                                 semi0)
                pltpu.make_async_copy(idx_hbm.at[base_blk], idx_v.at[0],
                                      semi0).wait()
                pltpu.async_copy(table_hbm.at[idx_v.at[0, 0]], rows0, sem0)

            pltpu.make_async_copy(table_hbm.at[idx_v.at[1, 0]],
                                  rows1, sem1).wait()
            pltpu.sync_copy(rows1, acc_sh.at[idx_v.at[1, 1]], add=True)

            @pl.when(b0 + 3 < nblk)
            def _():
                pltpu.async_copy(idx_hbm.at[base_blk + b0 + 3], idx_v.at[1],
                                 semi1)

        plsc.subcore_barrier()
        pltpu.sync_copy(acc_sh.at[pl.ds(sid * _RPT, _RPT)],
                        main_out.at[cid, pl.ds(sid * _RPT, _RPT)])

    return hop


def _make_rprop(nblk):
    @functools.partial(
        pl.kernel,
        out_type=jax.ShapeDtypeStruct((_NCORES, _NTILES, _NPAD), jnp.float32),
        mesh=_mesh(),
        scratch_types=[
            pltpu.VMEM((2, 2, _K), jnp.int32),
            pltpu.VMEM((_NPAD,), jnp.float32),
            pltpu.VMEM((_NPAD,), jnp.float32),
            pltpu.SemaphoreType.DMA,
            pltpu.SemaphoreType.DMA,
        ],
        compiler_params=_sc_params(),
    )
    def rprop(dinv_hbm, idx_hbm, out_hbm, idx_v, dinv_v, racc_v,
              semi0, semi1):
        cid = lax.axis_index("c")
        sid = lax.axis_index("s")
        pltpu.sync_copy(dinv_hbm, dinv_v)

        @pl.loop(0, _NPAD // 16)
        def _(i):
            racc_v[pl.ds(i * 16, 16)] = jnp.zeros((16,), jnp.float32)

        base_blk = (cid * _NTILES + sid) * nblk

        def work(p):
            @pl.loop(0, _K // 16)
            def _(j):
                s16 = idx_v[p, 0, pl.ds(j * 16, 16)]
                d16 = idx_v[p, 1, pl.ds(j * 16, 16)]
                vals = plsc.load_gather(dinv_v, [s16])
                plsc.addupdate_scatter(racc_v, [d16], vals)

        pltpu.sync_copy(idx_hbm.at[base_blk], idx_v.at[0])
        pltpu.async_copy(idx_hbm.at[base_blk + 1], idx_v.at[1], semi1)

        @pl.loop(0, nblk // 2)
        def _(i):
            b0 = i * 2
            work(0)

            @pl.when(b0 + 2 < nblk)
            def _():
                pltpu.async_copy(idx_hbm.at[base_blk + b0 + 2], idx_v.at[0],
                                 semi0)

            pltpu.make_async_copy(idx_hbm.at[base_blk], idx_v.at[1],
                                  semi1).wait()
            work(1)

            @pl.when(b0 + 3 < nblk)
            def _():
                pltpu.async_copy(idx_hbm.at[base_blk + b0 + 3], idx_v.at[1],
                                 semi1)

            @pl.when(b0 + 2 < nblk)
            def _():
                pltpu.make_async_copy(idx_hbm.at[base_blk], idx_v.at[0],
                                      semi0).wait()

        pltpu.sync_copy(racc_v, out_hbm.at[cid, sid])

    return rprop


def _make_hist(nblk):
    @functools.partial(
        pl.kernel,
        out_type=jax.ShapeDtypeStruct((_NCORES, _NTILES, _NPAD), jnp.float32),
        mesh=_mesh(),
        scratch_types=[
            pltpu.VMEM((2, 2, _K), jnp.int32),
            pltpu.VMEM((_NPAD,), jnp.float32),
            pltpu.SemaphoreType.DMA,
            pltpu.SemaphoreType.DMA,
        ],
        compiler_params=_sc_params(),
    )
    def hist(idx_hbm, out_hbm, idx_v, deg_v, semi0, semi1):
        cid = lax.axis_index("c")
        sid = lax.axis_index("s")

        @pl.loop(0, _NPAD // 16)
        def _(i):
            deg_v[pl.ds(i * 16, 16)] = jnp.zeros((16,), jnp.float32)

        base_blk = (cid * _NTILES + sid) * nblk
        ones16 = jnp.full((16,), 1.0, jnp.float32)

        def work(p):
            @pl.loop(0, _K // 16)
            def _(j):
                d16 = idx_v[p, 1, pl.ds(j * 16, 16)]
                plsc.addupdate_scatter(deg_v, [d16], ones16)

        pltpu.sync_copy(idx_hbm.at[base_blk], idx_v.at[0])
        pltpu.async_copy(idx_hbm.at[base_blk + 1], idx_v.at[1], semi1)

        @pl.loop(0, nblk // 2)
        def _(i):
            b0 = i * 2
            work(0)

            @pl.when(b0 + 2 < nblk)
            def _():
                pltpu.async_copy(idx_hbm.at[base_blk + b0 + 2], idx_v.at[0],
                                 semi0)

            pltpu.make_async_copy(idx_hbm.at[base_blk], idx_v.at[1],
                                  semi1).wait()
            work(1)

            @pl.when(b0 + 3 < nblk)
            def _():
                pltpu.async_copy(idx_hbm.at[base_blk + b0 + 3], idx_v.at[1],
                                 semi1)

            @pl.when(b0 + 2 < nblk)
            def _():
                pltpu.make_async_copy(idx_hbm.at[base_blk], idx_v.at[0],
                                      semi0).wait()

        pltpu.sync_copy(deg_v, out_hbm.at[cid, sid])

    return hist


def _final_tc_kernel(z_ref, r_ref, w0_ref, w1_ref, wout_ref, b0_ref, b1_ref,
                     bout_ref, vn_ref, out_ref):
    hi = jax.lax.Precision.HIGHEST
    w1wout = jax.lax.dot_general(
        w1_ref[...], wout_ref[...], (((1,), (0,)), ((), ())),
        preferred_element_type=jnp.float32, precision=hi)
    wc = jax.lax.dot_general(
        w0_ref[...], w1wout, (((1,), (0,)), ((), ())),
        preferred_element_type=jnp.float32, precision=hi)
    c0 = b0_ref[...] + vn_ref[...]
    cvec = jax.lax.dot_general(c0, w1wout, (((1,), (0,)), ((), ())),
                               preferred_element_type=jnp.float32, precision=hi)
    bvec = jax.lax.dot_general(b1_ref[...], wout_ref[...],
                               (((1,), (0,)), ((), ())),
                               preferred_element_type=jnp.float32,
                               precision=hi) + bout_ref[...]
    y = jax.lax.dot_general(z_ref[...], wc, (((1,), (0,)), ((), ())),
                            preferred_element_type=jnp.float32, precision=hi)
    out_ref[...] = y + r_ref[...] * cvec + bvec


def _final_matmul(z, r, W0, W1, Wout, b0, b1, bout, vn):
    n, d_out = z.shape[0], Wout.shape[1]
    return pl.pallas_call(
        _final_tc_kernel,
        out_shape=jax.ShapeDtypeStruct((n, d_out), jnp.float32),
    )(z, r.reshape(n, 1), W0, W1, Wout, b0.reshape(1, -1), b1.reshape(1, -1),
      bout.reshape(1, -1), vn)


def kernel(x, edge_index, W0, b0, W1, b1, vn, mW1, mb1, mW2, mb2, Wout, bout):
    n, d = x.shape
    src, dst = edge_index[0], edge_index[1]
    e = src.shape[0]
    nw = _NCORES * _NTILES
    nblk = -(-e // (nw * _K))
    nblk += nblk % 2
    epw = nblk * _K
    npad_edges = nw * epw - e
    pad_idx = (n + (jnp.arange(npad_edges) % (_NPAD - n))).astype(jnp.int32)
    srcs = jnp.concatenate([src, pad_idx])
    dsts = jnp.concatenate([dst, pad_idx])
    idx3 = jnp.stack([srcs.reshape(-1, _K), dsts.reshape(-1, _K)], axis=1)
    zero_tab = jnp.zeros((_NPAD, _D), jnp.float32)

    parts = _make_hist(nblk)(idx3)
    deg = jnp.sum(parts, axis=(0, 1))[:n] + 1.0
    dinv = lax.rsqrt(deg)

    dinv_pad = jnp.zeros((_NPAD,), jnp.float32).at[:n].set(dinv)
    u = x * dinv[:, None]
    u_pad = jnp.zeros((_NPAD, d), jnp.float32).at[:n].set(u)
    p1 = _make_hop(nblk, _D)(u_pad, idx3, zero_tab)
    rparts = _make_rprop(nblk)(dinv_pad, idx3)
    m = p1[0, :n] + p1[1, :n] + u
    r = dinv * (jnp.sum(rparts, axis=(0, 1))[:n] + dinv)
    w = m * (dinv * dinv)[:, None]

    w_pad = jnp.zeros((_NPAD, d), jnp.float32).at[:n].set(w)
    p2 = _make_hop(nblk, _D)(w_pad, idx3, zero_tab)
    z = (p2[0, :n] + p2[1, :n] + w) * dinv[:, None]

    return _final_matmul(z, r, W0, W1, Wout, b0, b1, bout, vn)


# speedup vs baseline: 11.4572x; 1.0155x over previous
import dataclasses
import functools

import jax
import jax.numpy as jnp
from jax import lax
from jax.experimental import pallas as pl
from jax.experimental.pallas import tpu as pltpu
from jax.experimental.pallas import tpu_sc as plsc

_NPAD = 10240
_NTILES = 16
_NCORES = 2
_RPT = _NPAD // _NTILES
_K = 128
_D = 128


def _mesh():
    return plsc.VectorSubcoreMesh(core_axis_name="c", subcore_axis_name="s")


def _sc_params():
    cp = pltpu.CompilerParams()
    if "needs_layout_passes" in pltpu.CompilerParams.__dataclass_fields__:
        cp = dataclasses.replace(cp, needs_layout_passes=False)
    return cp


def _make_hop(nblk, width):
    assert nblk % 2 == 0

    @functools.partial(
        pl.kernel,
        out_type=jax.ShapeDtypeStruct((_NCORES, _NPAD, width), jnp.float32),
        mesh=_mesh(),
        scratch_types=[
            pltpu.VMEM((2, 2, _K), jnp.int32),
            pltpu.VMEM((_K, width), jnp.float32),
            pltpu.VMEM((_K, width), jnp.float32),
            pltpu.VMEM_SHARED((_NPAD, width), jnp.float32),
            pltpu.SemaphoreType.DMA,
            pltpu.SemaphoreType.DMA,
            pltpu.SemaphoreType.DMA,
            pltpu.SemaphoreType.DMA,
            pltpu.SemaphoreType.DMA,
        ],
        compiler_params=_sc_params(),
    )
    def hop(table_hbm, idx_hbm, zero_hbm, main_out, idx_v, rows0, rows1,
            acc_sh, sem0, sem1, semi0, semi1, semz):
        cid = lax.axis_index("c")
        sid = lax.axis_index("s")
        pltpu.async_copy(zero_hbm.at[pl.ds(sid * _RPT, _RPT)],
                         acc_sh.at[pl.ds(sid * _RPT, _RPT)], semz)
        base_blk = (cid * _NTILES + sid) * nblk

        pltpu.sync_copy(idx_hbm.at[base_blk], idx_v.at[0])
        pltpu.async_copy(idx_hbm.at[base_blk + 1], idx_v.at[1], semi1)
        pltpu.async_copy(table_hbm.at[idx_v.at[0, 0]], rows0, sem0)
        pltpu.make_async_copy(zero_hbm.at[pl.ds(sid * _RPT, _RPT)],
                              acc_sh.at[pl.ds(sid * _RPT, _RPT)], semz).wait()
        plsc.subcore_barrier()

        @pl.loop(0, nblk // 2)
        def _(i):
            b0 = i * 2
            pltpu.make_async_copy(idx_hbm.at[base_blk], idx_v.at[1],
                                  semi1).wait()
            pltpu.async_copy(table_hbm.at[idx_v.at[1, 0]], rows1, sem1)
            pltpu.make_async_copy(table_hbm.at[idx_v.at[0, 0]],
                                  rows0, sem0).wait()
            pltpu.sync_copy(rows0, acc_sh.at[idx_v.at[0, 1]], add=True)

            @pl.when(b0 + 2 < nblk)
            def _():
                pltpu.async_copy(idx_hbm.at[base_blk + b0 + 2], idx_v.at[0],
                                 semi0)
                pltpu.make_async_copy(idx_hbm.at[base_blk], idx_v.at[0],
                                      semi0).wait()
                pltpu.async_copy(table_hbm.at[idx_v.at[0, 0]], rows0, sem0)

            pltpu.make_async_copy(table_hbm.at[idx_v.at[1, 0]],
                                  rows1, sem1).wait()
            pltpu.sync_copy(rows1, acc_sh.at[idx_v.at[1, 1]], add=True)

            @pl.when(b0 + 3 < nblk)
            def _():
                pltpu.async_copy(idx_hbm.at[base_blk + b0 + 3], idx_v.at[1],
                                 semi1)

        plsc.subcore_barrier()
        pltpu.sync_copy(acc_sh.at[pl.ds(sid * _RPT, _RPT)],
                        main_out.at[cid, pl.ds(sid * _RPT, _RPT)])

    return hop


def _make_rprop(nblk):
    @functools.partial(
        pl.kernel,
        out_type=jax.ShapeDtypeStruct((_NCORES, _NTILES, _NPAD), jnp.float32),
        mesh=_mesh(),
        scratch_types=[
            pltpu.VMEM((2, 2, _K), jnp.int32),
            pltpu.VMEM((_NPAD,), jnp.float32),
            pltpu.VMEM((_NPAD,), jnp.float32),
            pltpu.SemaphoreType.DMA,
            pltpu.SemaphoreType.DMA,
        ],
        compiler_params=_sc_params(),
    )
    def rprop(dinv_hbm, idx_hbm, out_hbm, idx_v, dinv_v, racc_v,
              semi0, semi1):
        cid = lax.axis_index("c")
        sid = lax.axis_index("s")
        pltpu.sync_copy(dinv_hbm, dinv_v)

        @pl.loop(0, _NPAD // 16)
        def _(i):
            racc_v[pl.ds(i * 16, 16)] = jnp.zeros((16,), jnp.float32)

        base_blk = (cid * _NTILES + sid) * nblk

        def work(p):
            @pl.loop(0, _K // 16)
            def _(j):
                s16 = idx_v[p, 0, pl.ds(j * 16, 16)]
                d16 = idx_v[p, 1, pl.ds(j * 16, 16)]
                vals = plsc.load_gather(dinv_v, [s16])
                plsc.addupdate_scatter(racc_v, [d16], vals)

        pltpu.sync_copy(idx_hbm.at[base_blk], idx_v.at[0])
        pltpu.async_copy(idx_hbm.at[base_blk + 1], idx_v.at[1], semi1)

        @pl.loop(0, nblk // 2)
        def _(i):
            b0 = i * 2
            work(0)

            @pl.when(b0 + 2 < nblk)
            def _():
                pltpu.async_copy(idx_hbm.at[base_blk + b0 + 2], idx_v.at[0],
                                 semi0)

            pltpu.make_async_copy(idx_hbm.at[base_blk], idx_v.at[1],
                                  semi1).wait()
            work(1)

            @pl.when(b0 + 3 < nblk)
            def _():
                pltpu.async_copy(idx_hbm.at[base_blk + b0 + 3], idx_v.at[1],
                                 semi1)

            @pl.when(b0 + 2 < nblk)
            def _():
                pltpu.make_async_copy(idx_hbm.at[base_blk], idx_v.at[0],
                                      semi0).wait()

        pltpu.sync_copy(racc_v, out_hbm.at[cid, sid])

    return rprop


def _make_hist(nblk):
    @functools.partial(
        pl.kernel,
        out_type=jax.ShapeDtypeStruct((_NCORES, _NTILES, _NPAD), jnp.float32),
        mesh=_mesh(),
        scratch_types=[
            pltpu.VMEM((2, 2, _K), jnp.int32),
            pltpu.VMEM((_NPAD,), jnp.float32),
            pltpu.SemaphoreType.DMA,
            pltpu.SemaphoreType.DMA,
        ],
        compiler_params=_sc_params(),
    )
    def hist(idx_hbm, out_hbm, idx_v, deg_v, semi0, semi1):
        cid = lax.axis_index("c")
        sid = lax.axis_index("s")

        @pl.loop(0, _NPAD // 16)
        def _(i):
            deg_v[pl.ds(i * 16, 16)] = jnp.zeros((16,), jnp.float32)

        base_blk = (cid * _NTILES + sid) * nblk
        ones16 = jnp.full((16,), 1.0, jnp.float32)

        def work(p):
            @pl.loop(0, _K // 16)
            def _(j):
                d16 = idx_v[p, 1, pl.ds(j * 16, 16)]
                plsc.addupdate_scatter(deg_v, [d16], ones16)

        pltpu.sync_copy(idx_hbm.at[base_blk], idx_v.at[0])
        pltpu.async_copy(idx_hbm.at[base_blk + 1], idx_v.at[1], semi1)

        @pl.loop(0, nblk // 2)
        def _(i):
            b0 = i * 2
            work(0)

            @pl.when(b0 + 2 < nblk)
            def _():
                pltpu.async_copy(idx_hbm.at[base_blk + b0 + 2], idx_v.at[0],
                                 semi0)

            pltpu.make_async_copy(idx_hbm.at[base_blk], idx_v.at[1],
                                  semi1).wait()
            work(1)

            @pl.when(b0 + 3 < nblk)
            def _():
                pltpu.async_copy(idx_hbm.at[base_blk + b0 + 3], idx_v.at[1],
                                 semi1)

            @pl.when(b0 + 2 < nblk)
            def _():
                pltpu.make_async_copy(idx_hbm.at[base_blk], idx_v.at[0],
                                      semi0).wait()

        pltpu.sync_copy(deg_v, out_hbm.at[cid, sid])

    return hist


def _final_tc_kernel(p2_ref, w_ref, dinv_ref, r_ref, w0_ref, w1_ref,
                     wout_ref, b0_ref, b1_ref, bout_ref, vn_ref, out_ref):
    hi = jax.lax.Precision.HIGHEST
    nn = w_ref.shape[0]
    z = (p2_ref[0, :nn] + p2_ref[1, :nn] + w_ref[...]) * dinv_ref[...]
    w1wout = jax.lax.dot_general(
        w1_ref[...], wout_ref[...], (((1,), (0,)), ((), ())),
        preferred_element_type=jnp.float32, precision=hi)
    wc = jax.lax.dot_general(
        w0_ref[...], w1wout, (((1,), (0,)), ((), ())),
        preferred_element_type=jnp.float32, precision=hi)
    c0 = b0_ref[...] + vn_ref[...]
    cvec = jax.lax.dot_general(c0, w1wout, (((1,), (0,)), ((), ())),
                               preferred_element_type=jnp.float32, precision=hi)
    bvec = jax.lax.dot_general(b1_ref[...], wout_ref[...],
                               (((1,), (0,)), ((), ())),
                               preferred_element_type=jnp.float32,
                               precision=hi) + bout_ref[...]
    y = jax.lax.dot_general(z, wc, (((1,), (0,)), ((), ())),
                            preferred_element_type=jnp.float32, precision=hi)
    out_ref[...] = y + r_ref[...] * cvec + bvec


def _final_matmul(p2, w, dinv, r, W0, W1, Wout, b0, b1, bout, vn):
    n, d_out = w.shape[0], Wout.shape[1]
    return pl.pallas_call(
        _final_tc_kernel,
        out_shape=jax.ShapeDtypeStruct((n, d_out), jnp.float32),
    )(p2, w, dinv.reshape(n, 1), r.reshape(n, 1), W0, W1, Wout,
      b0.reshape(1, -1), b1.reshape(1, -1), bout.reshape(1, -1), vn)


def kernel(x, edge_index, W0, b0, W1, b1, vn, mW1, mb1, mW2, mb2, Wout, bout):
    n, d = x.shape
    src, dst = edge_index[0], edge_index[1]
    e = src.shape[0]
    nw = _NCORES * _NTILES
    nblk = -(-e // (nw * _K))
    nblk += nblk % 2
    epw = nblk * _K
    npad_edges = nw * epw - e
    pad_idx = (n + (jnp.arange(npad_edges) % (_NPAD - n))).astype(jnp.int32)
    srcs = jnp.concatenate([src, pad_idx])
    dsts = jnp.concatenate([dst, pad_idx])
    idx3 = jnp.stack([srcs.reshape(-1, _K), dsts.reshape(-1, _K)], axis=1)
    zero_tab = jnp.zeros((_NPAD, _D), jnp.float32)

    parts = _make_hist(nblk)(idx3)
    deg = jnp.sum(parts, axis=(0, 1))[:n] + 1.0
    dinv = lax.rsqrt(deg)

    dinv_pad = jnp.zeros((_NPAD,), jnp.float32).at[:n].set(dinv)
    u = x * dinv[:, None]
    u_pad = jnp.zeros((_NPAD, d), jnp.float32).at[:n].set(u)
    p1 = _make_hop(nblk, _D)(u_pad, idx3, zero_tab)
    rparts = _make_rprop(nblk)(dinv_pad, idx3)
    m = p1[0, :n] + p1[1, :n] + u
    r = dinv * (jnp.sum(rparts, axis=(0, 1))[:n] + dinv)
    w = m * (dinv * dinv)[:, None]

    w_pad = jnp.zeros((_NPAD, d), jnp.float32).at[:n].set(w)
    p2 = _make_hop(nblk, _D)(w_pad, idx3, zero_tab)
    return _final_matmul(p2, w, dinv, r, W0, W1, Wout, b0, b1, bout, vn)


# speedup vs baseline: 12.4637x; 1.0878x over previous
import dataclasses
import functools

import jax
import jax.numpy as jnp
from jax import lax
from jax.experimental import pallas as pl
from jax.experimental.pallas import tpu as pltpu
from jax.experimental.pallas import tpu_sc as plsc

_NPAD = 10240
_NTILES = 16
_NCORES = 2
_RPT = _NPAD // _NTILES
_K = 128
_D = 128


def _mesh():
    return plsc.VectorSubcoreMesh(core_axis_name="c", subcore_axis_name="s")


def _sc_params():
    cp = pltpu.CompilerParams()
    if "needs_layout_passes" in pltpu.CompilerParams.__dataclass_fields__:
        cp = dataclasses.replace(cp, needs_layout_passes=False)
    return cp


def _make_hop(nblk, width):
    assert nblk % 2 == 0

    @functools.partial(
        pl.kernel,
        out_type=jax.ShapeDtypeStruct((_NCORES, _NPAD, width), jnp.float32),
        mesh=_mesh(),
        scratch_types=[
            pltpu.VMEM((2, 2, _K), jnp.int32),
            pltpu.VMEM((_K, width), jnp.float32),
            pltpu.VMEM((_K, width), jnp.float32),
            pltpu.VMEM_SHARED((_NPAD, width), jnp.float32),
            pltpu.SemaphoreType.DMA,
            pltpu.SemaphoreType.DMA,
            pltpu.SemaphoreType.DMA,
            pltpu.SemaphoreType.DMA,
            pltpu.SemaphoreType.DMA,
            pltpu.SemaphoreType.DMA,
            pltpu.SemaphoreType.DMA,
        ],
        compiler_params=_sc_params(),
    )
    def hop(table_hbm, idx_hbm, zero_hbm, main_out, idx_v, rows0, rows1,
            acc_sh, sem0, sem1, semi0, semi1, semz, semsc0, semsc1):
        cid = lax.axis_index("c")
        sid = lax.axis_index("s")
        pltpu.async_copy(zero_hbm.at[pl.ds(sid * _RPT, _RPT)],
                         acc_sh.at[pl.ds(sid * _RPT, _RPT)], semz)
        base_blk = (cid * _NTILES + sid) * nblk

        pltpu.sync_copy(idx_hbm.at[base_blk], idx_v.at[0])
        pltpu.async_copy(idx_hbm.at[base_blk + 1], idx_v.at[1], semi1)
        pltpu.async_copy(table_hbm.at[idx_v.at[0, 0]], rows0, sem0)
        pltpu.make_async_copy(zero_hbm.at[pl.ds(sid * _RPT, _RPT)],
                              acc_sh.at[pl.ds(sid * _RPT, _RPT)], semz).wait()
        plsc.subcore_barrier()

        @pl.loop(0, nblk // 2)
        def _(i):
            b0 = i * 2
            pltpu.make_async_copy(idx_hbm.at[base_blk], idx_v.at[1],
                                  semi1).wait()

            @pl.when(i > 0)
            def _():
                pltpu.make_async_copy(rows1, acc_sh.at[idx_v.at[1, 1]],
                                      semsc1).wait()

            pltpu.async_copy(table_hbm.at[idx_v.at[1, 0]], rows1, sem1)
            pltpu.make_async_copy(table_hbm.at[idx_v.at[0, 0]],
                                  rows0, sem0).wait()
            pltpu.async_copy(rows0, acc_sh.at[idx_v.at[0, 1]], semsc0,
                             add=True)

            @pl.when(b0 + 2 < nblk)
            def _():
                pltpu.async_copy(idx_hbm.at[base_blk + b0 + 2], idx_v.at[0],
                                 semi0)
                pltpu.make_async_copy(idx_hbm.at[base_blk], idx_v.at[0],
                                      semi0).wait()
                pltpu.make_async_copy(rows0, acc_sh.at[idx_v.at[0, 1]],
                                      semsc0).wait()
                pltpu.async_copy(table_hbm.at[idx_v.at[0, 0]], rows0, sem0)

            @pl.when(b0 + 2 >= nblk)
            def _():
                pltpu.make_async_copy(rows0, acc_sh.at[idx_v.at[0, 1]],
                                      semsc0).wait()

            pltpu.make_async_copy(table_hbm.at[idx_v.at[1, 0]],
                                  rows1, sem1).wait()
            pltpu.async_copy(rows1, acc_sh.at[idx_v.at[1, 1]], semsc1,
                             add=True)

            @pl.when(b0 + 3 < nblk)
            def _():
                pltpu.async_copy(idx_hbm.at[base_blk + b0 + 3], idx_v.at[1],
                                 semi1)

        pltpu.make_async_copy(rows1, acc_sh.at[idx_v.at[1, 1]],
                              semsc1).wait()
        plsc.subcore_barrier()
        pltpu.sync_copy(acc_sh.at[pl.ds(sid * _RPT, _RPT)],
                        main_out.at[cid, pl.ds(sid * _RPT, _RPT)])

    return hop


def _make_rprop(nblk):
    @functools.partial(
        pl.kernel,
        out_type=jax.ShapeDtypeStruct((_NCORES, _NTILES, _NPAD), jnp.float32),
        mesh=_mesh(),
        scratch_types=[
            pltpu.VMEM((2, 2, _K), jnp.int32),
            pltpu.VMEM((_NPAD,), jnp.float32),
            pltpu.VMEM((_NPAD,), jnp.float32),
            pltpu.SemaphoreType.DMA,
            pltpu.SemaphoreType.DMA,
        ],
        compiler_params=_sc_params(),
    )
    def rprop(dinv_hbm, idx_hbm, out_hbm, idx_v, dinv_v, racc_v,
              semi0, semi1):
        cid = lax.axis_index("c")
        sid = lax.axis_index("s")
        pltpu.sync_copy(dinv_hbm, dinv_v)

        @pl.loop(0, _NPAD // 16)
        def _(i):
            racc_v[pl.ds(i * 16, 16)] = jnp.zeros((16,), jnp.float32)

        base_blk = (cid * _NTILES + sid) * nblk

        def work(p):
            @pl.loop(0, _K // 16)
            def _(j):
                s16 = idx_v[p, 0, pl.ds(j * 16, 16)]
                d16 = idx_v[p, 1, pl.ds(j * 16, 16)]
                vals = plsc.load_gather(dinv_v, [s16])
                plsc.addupdate_scatter(racc_v, [d16], vals)

        pltpu.sync_copy(idx_hbm.at[base_blk], idx_v.at[0])
        pltpu.async_copy(idx_hbm.at[base_blk + 1], idx_v.at[1], semi1)

        @pl.loop(0, nblk // 2)
        def _(i):
            b0 = i * 2
            work(0)

            @pl.when(b0 + 2 < nblk)
            def _():
                pltpu.async_copy(idx_hbm.at[base_blk + b0 + 2], idx_v.at[0],
                                 semi0)

            pltpu.make_async_copy(idx_hbm.at[base_blk], idx_v.at[1],
                                  semi1).wait()
            work(1)

            @pl.when(b0 + 3 < nblk)
            def _():
                pltpu.async_copy(idx_hbm.at[base_blk + b0 + 3], idx_v.at[1],
                                 semi1)

            @pl.when(b0 + 2 < nblk)
            def _():
                pltpu.make_async_copy(idx_hbm.at[base_blk], idx_v.at[0],
                                      semi0).wait()

        pltpu.sync_copy(racc_v, out_hbm.at[cid, sid])

    return rprop


def _make_hist(nblk):
    @functools.partial(
        pl.kernel,
        out_type=jax.ShapeDtypeStruct((_NCORES, _NTILES, _NPAD), jnp.float32),
        mesh=_mesh(),
        scratch_types=[
            pltpu.VMEM((2, 2, _K), jnp.int32),
            pltpu.VMEM((_NPAD,), jnp.float32),
            pltpu.SemaphoreType.DMA,
            pltpu.SemaphoreType.DMA,
        ],
        compiler_params=_sc_params(),
    )
    def hist(idx_hbm, out_hbm, idx_v, deg_v, semi0, semi1):
        cid = lax.axis_index("c")
        sid = lax.axis_index("s")

        @pl.loop(0, _NPAD // 16)
        def _(i):
            deg_v[pl.ds(i * 16, 16)] = jnp.zeros((16,), jnp.float32)

        base_blk = (cid * _NTILES + sid) * nblk
        ones16 = jnp.full((16,), 1.0, jnp.float32)

        def work(p):
            @pl.loop(0, _K // 16)
            def _(j):
                d16 = idx_v[p, 1, pl.ds(j * 16, 16)]
                plsc.addupdate_scatter(deg_v, [d16], ones16)

        pltpu.sync_copy(idx_hbm.at[base_blk], idx_v.at[0])
        pltpu.async_copy(idx_hbm.at[base_blk + 1], idx_v.at[1], semi1)

        @pl.loop(0, nblk // 2)
        def _(i):
            b0 = i * 2
            work(0)

            @pl.when(b0 + 2 < nblk)
            def _():
                pltpu.async_copy(idx_hbm.at[base_blk + b0 + 2], idx_v.at[0],
                                 semi0)

            pltpu.make_async_copy(idx_hbm.at[base_blk], idx_v.at[1],
                                  semi1).wait()
            work(1)

            @pl.when(b0 + 3 < nblk)
            def _():
                pltpu.async_copy(idx_hbm.at[base_blk + b0 + 3], idx_v.at[1],
                                 semi1)

            @pl.when(b0 + 2 < nblk)
            def _():
                pltpu.make_async_copy(idx_hbm.at[base_blk], idx_v.at[0],
                                      semi0).wait()

        pltpu.sync_copy(deg_v, out_hbm.at[cid, sid])

    return hist


def _final_tc_kernel(p2_ref, w_ref, dinv_ref, r_ref, w0_ref, w1_ref,
                     wout_ref, b0_ref, b1_ref, bout_ref, vn_ref, out_ref):
    hi = jax.lax.Precision.HIGHEST
    nn = w_ref.shape[0]
    z = (p2_ref[0, :nn] + p2_ref[1, :nn] + w_ref[...]) * dinv_ref[...]
    w1wout = jax.lax.dot_general(
        w1_ref[...], wout_ref[...], (((1,), (0,)), ((), ())),
        preferred_element_type=jnp.float32, precision=hi)
    wc = jax.lax.dot_general(
        w0_ref[...], w1wout, (((1,), (0,)), ((), ())),
        preferred_element_type=jnp.float32, precision=hi)
    c0 = b0_ref[...] + vn_ref[...]
    cvec = jax.lax.dot_general(c0, w1wout, (((1,), (0,)), ((), ())),
                               preferred_element_type=jnp.float32, precision=hi)
    bvec = jax.lax.dot_general(b1_ref[...], wout_ref[...],
                               (((1,), (0,)), ((), ())),
                               preferred_element_type=jnp.float32,
                               precision=hi) + bout_ref[...]
    y = jax.lax.dot_general(z, wc, (((1,), (0,)), ((), ())),
                            preferred_element_type=jnp.float32, precision=hi)
    out_ref[...] = y + r_ref[...] * cvec + bvec


def _final_matmul(p2, w, dinv, r, W0, W1, Wout, b0, b1, bout, vn):
    n, d_out = w.shape[0], Wout.shape[1]
    return pl.pallas_call(
        _final_tc_kernel,
        out_shape=jax.ShapeDtypeStruct((n, d_out), jnp.float32),
    )(p2, w, dinv.reshape(n, 1), r.reshape(n, 1), W0, W1, Wout,
      b0.reshape(1, -1), b1.reshape(1, -1), bout.reshape(1, -1), vn)


def kernel(x, edge_index, W0, b0, W1, b1, vn, mW1, mb1, mW2, mb2, Wout, bout):
    n, d = x.shape
    src, dst = edge_index[0], edge_index[1]
    e = src.shape[0]
    nw = _NCORES * _NTILES
    nblk = -(-e // (nw * _K))
    nblk += nblk % 2
    epw = nblk * _K
    npad_edges = nw * epw - e
    pad_idx = (n + (jnp.arange(npad_edges) % (_NPAD - n))).astype(jnp.int32)
    srcs = jnp.concatenate([src, pad_idx])
    dsts = jnp.concatenate([dst, pad_idx])
    idx3 = jnp.stack([srcs.reshape(-1, _K), dsts.reshape(-1, _K)], axis=1)
    zero_tab = jnp.zeros((_NPAD, _D), jnp.float32)

    parts = _make_hist(nblk)(idx3)
    deg = jnp.sum(parts, axis=(0, 1))[:n] + 1.0
    dinv = lax.rsqrt(deg)

    dinv_pad = jnp.zeros((_NPAD,), jnp.float32).at[:n].set(dinv)
    u = x * dinv[:, None]
    u_pad = jnp.zeros((_NPAD, d), jnp.float32).at[:n].set(u)
    p1 = _make_hop(nblk, _D)(u_pad, idx3, zero_tab)
    rparts = _make_rprop(nblk)(dinv_pad, idx3)
    m = p1[0, :n] + p1[1, :n] + u
    r = dinv * (jnp.sum(rparts, axis=(0, 1))[:n] + dinv)
    w = m * (dinv * dinv)[:, None]

    w_pad = jnp.zeros((_NPAD, d), jnp.float32).at[:n].set(w)
    p2 = _make_hop(nblk, _D)(w_pad, idx3, zero_tab)
    return _final_matmul(p2, w, dinv, r, W0, W1, Wout, b0, b1, bout, vn)
